# baseline (speedup 1.0000x reference)
import functools

import jax
import jax.numpy as jnp
import numpy as np
from jax import lax
from jax.experimental import pallas as pl
from jax.experimental.pallas import tpu as pltpu
from jax.experimental.pallas import tpu_sc as plsc

_DIMS = (119, 5, 12, 12, 10, 6, 6, 2, 2)
_OFFS = tuple(int(o) for o in np.cumsum((0,) + _DIMS))
_F = _OFFS[-1]
_FP = 256
_TROWS = 176
_EMB = 128
_N = 100000
_BM = 5000

_NW = 32
_RPW = 3136
_CH = 8


def _idx_body(x_ref, idxw_ref, o_ref):
    xb = x_ref[...]
    parts = [
        jnp.broadcast_to(jnp.max(xb[:, o:o + d], axis=1, keepdims=True),
                         (_BM, d))
        for o, d in zip(_OFFS[:-1], _DIMS)
    ]
    mxmap = jnp.concatenate(parts, axis=1)
    eq = (xb == mxmap)
    ohb = jnp.concatenate(
        [eq.astype(jnp.bfloat16), jnp.zeros((_BM, _FP - _F), jnp.bfloat16)],
        axis=1)
    idxf = jax.lax.dot_general(ohb, idxw_ref[...], (((1,), (0,)), ((), ())),
                               preferred_element_type=jnp.float32)
    o_ref[...] = jnp.minimum(idxf[:, :9], float(_TROWS - 1)).astype(jnp.int32)


def _sc_lookup(gidx_flat, tbl):
    mesh = plsc.VectorSubcoreMesh(core_axis_name="c", subcore_axis_name="s")

    @functools.partial(
        pl.kernel, mesh=mesh,
        out_type=jax.ShapeDtypeStruct((_N, _EMB), jnp.float32),
        scratch_types=[
            pltpu.VMEM((_CH * 9,), jnp.int32),
            pltpu.VMEM((_CH * 9, _EMB), jnp.float32),
            pltpu.VMEM((_CH, _EMB), jnp.float32),
            pltpu.SemaphoreType.DMA,
        ],
    )
    def k(gidx_hbm, tbl_hbm, out_hbm, idx_v, rows_v, out_v, sem):
        wid = lax.axis_index("s") * 2 + lax.axis_index("c")
        base = wid * _RPW
        nch = jnp.where(wid < _NW - 1, _RPW // _CH, (_N - (_NW - 1) * _RPW) // _CH)

        def chunk(kk, carry):
            r0 = base + _CH * kk
            pltpu.sync_copy(gidx_hbm.at[pl.ds(9 * r0, 9 * _CH)], idx_v)
            pltpu.async_copy(tbl_hbm.at[idx_v], rows_v, sem).wait()
            for r in range(_CH):
                for c in range(_EMB // 16):
                    acc = rows_v[9 * r, pl.ds(16 * c, 16)]
                    for j in range(1, 9):
                        acc = acc + rows_v[9 * r + j, pl.ds(16 * c, 16)]
                    out_v[r, pl.ds(16 * c, 16)] = acc
            pltpu.sync_copy(out_v, out_hbm.at[pl.ds(r0, _CH)])
            return carry

        lax.fori_loop(0, nch, chunk, 0)

    return k(gidx_flat, tbl)


@jax.jit
def kernel(x, W0, W1, W2, W3, W4, W5, W6, W7, W8):
    tbl = jnp.concatenate([W0, W1, W2, W3, W4, W5, W6, W7, W8], axis=0)
    tbl = jnp.pad(tbl, ((0, _TROWS - _F), (0, 0)))

    idxw = np.zeros((_FP, _EMB), np.float32)
    for g, (o, d) in enumerate(zip(_OFFS[:-1], _DIMS)):
        idxw[o:o + d, g] = np.arange(o, o + d, dtype=np.float32)
    idxw = jnp.asarray(idxw, dtype=jnp.bfloat16)

    gidx = pl.pallas_call(
        _idx_body,
        grid=(_N // _BM,),
        in_specs=[
            pl.BlockSpec((_BM, _F), lambda i: (i, 0)),
            pl.BlockSpec((_FP, _EMB), lambda i: (0, 0)),
        ],
        out_specs=pl.BlockSpec((_BM, 9), lambda i: (i, 0)),
        out_shape=jax.ShapeDtypeStruct((_N, 9), jnp.int32),
    )(x, idxw)

    return _sc_lookup(gidx.reshape(_N * 9), tbl)
